# Initial kernel scaffold; baseline (speedup 1.0000x reference)
import functools

import jax
import jax.numpy as jnp
from jax import lax
from jax.experimental import pallas as pl
from jax.experimental.pallas import tpu as pltpu
from jax.experimental.pallas import tpu_sc as plsc

B = 4
T = 2048
D = 1024
L = 16
NC = 2
NS = 16
NW = NC * NS
PER = (B * T) // NW
VPT = PER // L
VPR = T // L
R = 64
NCH = PER // R


def _body(inp_hbm, tab_hbm, out_hbm, row_v, chunk_v, idx_v, buf_v, sem):
    wid = lax.axis_index("s") * NC + lax.axis_index("c")
    row = wid // (T // PER)
    ch = wid % (T // PER)

    pltpu.sync_copy(inp_hbm.at[pl.ds(row * T, T)], row_v)
    pltpu.sync_copy(inp_hbm.at[pl.ds(wid * PER, PER)], chunk_v)

    nvec = jnp.broadcast_to(ch * (PER // L), (L,))
    sumvec = jnp.zeros((L,), jnp.int32)
    for j in range(VPR):
        v = row_v[pl.ds(j * L, L)]
        m = (v != 0).astype(jnp.int32)
        take = jnp.broadcast_to(jnp.int32(j), (L,)) < nvec
        sumvec = sumvec + jnp.where(take, m, 0)
    prefix = jnp.broadcast_to(jnp.sum(sumvec), (L,))

    carry = prefix
    for j in range(VPT):
        v = chunk_v[pl.ds(j * L, L)]
        m = v != 0
        mi = m.astype(jnp.int32)
        pos = (carry + plsc.cumsum(mi)) * mi
        idx_v[j // (R // L), pl.ds((j % (R // L)) * L, L)] = pos
        carry = carry + plsc.all_reduce_population_count(m)

    base = wid * PER
    for k in range(NCH):
        pltpu.async_copy(tab_hbm.at[idx_v.at[k]], buf_v, sem).wait()
        pltpu.sync_copy(buf_v, out_hbm.at[pl.ds(base + k * R, R)])


@jax.jit
def _sc_embed(flat_inp, table):
    mesh = plsc.VectorSubcoreMesh(
        core_axis_name="c", subcore_axis_name="s", num_cores=NC, num_subcores=NS
    )
    return pl.kernel(
        _body,
        out_type=jax.ShapeDtypeStruct((B * T, D), jnp.float32),
        mesh=mesh,
        scratch_types=[
            pltpu.VMEM((T,), jnp.int32),
            pltpu.VMEM((PER,), jnp.int32),
            pltpu.VMEM((NCH, R), jnp.int32),
            pltpu.VMEM((R, D), jnp.float32),
            pltpu.SemaphoreType.DMA,
        ],
    )(flat_inp, table)


def kernel(input, embeddings):
    flat = input.reshape(-1).astype(jnp.int32)
    out = _sc_embed(flat, embeddings.astype(jnp.float32))
    return out.reshape(B, T, D)


# speedup vs baseline: 1.4441x; 1.4441x over previous
import functools

import jax
import jax.numpy as jnp
from jax import lax
from jax.experimental import pallas as pl
from jax.experimental.pallas import tpu as pltpu
from jax.experimental.pallas import tpu_sc as plsc

B = 4
T = 2048
D = 1024
L = 16
NC = 2
NS = 16
NW = NC * NS
PER = (B * T) // NW
VPT = PER // L
VPR = T // L
R = 64
NCH = PER // R


def _body(inp_hbm, tab_hbm, out_hbm, row_v, chunk_v, idx_v, buf_v, sem):
    wid = lax.axis_index("s") * NC + lax.axis_index("c")
    row = wid // (T // PER)
    ch = wid % (T // PER)

    pltpu.sync_copy(inp_hbm.at[pl.ds(row * T, T)], row_v)
    pltpu.sync_copy(inp_hbm.at[pl.ds(wid * PER, PER)], chunk_v)

    zeros = jnp.zeros((L,), jnp.int32)
    lane = lax.iota(jnp.int32, L)
    last = jnp.broadcast_to(jnp.int32(L - 1), (L,))

    def nonzero_mask(v):
        return lax.shift_right_logical(v | (zeros - v), 31)

    def scan16(x):
        s = x
        for d in (1, 2, 4, 8):
            shifted = s.at[jnp.maximum(lane - d, 0)].get(mode="promise_in_bounds")
            s = s + jnp.clip(lane - (d - 1), 0, 1) * shifted
        return s

    def splat_last(s):
        return s.at[last].get(mode="promise_in_bounds")

    nvec = jnp.broadcast_to(ch * (PER // L), (L,))
    sumvec = zeros
    for j in range(VPR):
        v = row_v[pl.ds(j * L, L)]
        take = jnp.clip(nvec - jnp.int32(j), 0, 1)
        sumvec = sumvec + take * nonzero_mask(v)
    prefix = splat_last(scan16(sumvec))

    carry = prefix
    for j in range(VPT):
        v = chunk_v[pl.ds(j * L, L)]
        mi = nonzero_mask(v)
        s = scan16(mi)
        pos = (carry + s) * mi
        idx_v[j // (R // L), pl.ds((j % (R // L)) * L, L)] = pos
        carry = carry + splat_last(s)

    base = wid * PER
    for k in range(NCH):
        pltpu.async_copy(tab_hbm.at[idx_v.at[k]], buf_v, sem).wait()
        pltpu.sync_copy(buf_v, out_hbm.at[pl.ds(base + k * R, R)])


@jax.jit
def _sc_embed(flat_inp, table):
    mesh = plsc.VectorSubcoreMesh(
        core_axis_name="c", subcore_axis_name="s", num_cores=NC, num_subcores=NS
    )
    return pl.kernel(
        _body,
        out_type=jax.ShapeDtypeStruct((B * T, D), jnp.float32),
        mesh=mesh,
        scratch_types=[
            pltpu.VMEM((T,), jnp.int32),
            pltpu.VMEM((PER,), jnp.int32),
            pltpu.VMEM((NCH, R), jnp.int32),
            pltpu.VMEM((R, D), jnp.float32),
            pltpu.SemaphoreType.DMA,
        ],
    )(flat_inp, table)


def kernel(input, embeddings):
    flat = input.reshape(-1).astype(jnp.int32)
    out = _sc_embed(flat, embeddings.astype(jnp.float32))
    return out.reshape(B, T, D)


# speedup vs baseline: 1.4693x; 1.0174x over previous
import functools

import jax
import jax.numpy as jnp
from jax import lax
from jax.experimental import pallas as pl
from jax.experimental.pallas import tpu as pltpu
from jax.experimental.pallas import tpu_sc as plsc

B = 4
T = 2048
D = 1024
L = 16
NC = 2
NS = 16
NW = NC * NS
PER = (B * T) // NW
VPT = PER // L
VPR = T // L
R = 32
NCH = PER // R
NB = 2


def _body(inp_hbm, tab_hbm, out_hbm, row_v, chunk_v, idx_v, buf_v,
          gsem0, gsem1, osem0, osem1):
    wid = lax.axis_index("s") * NC + lax.axis_index("c")
    row = wid // (T // PER)
    ch = wid % (T // PER)

    pltpu.sync_copy(inp_hbm.at[pl.ds(row * T, T)], row_v)
    pltpu.sync_copy(inp_hbm.at[pl.ds(wid * PER, PER)], chunk_v)

    zeros = jnp.zeros((L,), jnp.int32)
    lane = lax.iota(jnp.int32, L)
    last = jnp.broadcast_to(jnp.int32(L - 1), (L,))

    def nonzero_mask(v):
        return lax.shift_right_logical(v | (zeros - v), 31)

    def scan16(x):
        s = x
        for d in (1, 2, 4, 8):
            shifted = s.at[jnp.maximum(lane - d, 0)].get(mode="promise_in_bounds")
            s = s + jnp.clip(lane - (d - 1), 0, 1) * shifted
        return s

    def splat_last(s):
        return s.at[last].get(mode="promise_in_bounds")

    nvec = jnp.broadcast_to(ch * (PER // L), (L,))
    sumvec = zeros
    for j in range(VPR):
        v = row_v[pl.ds(j * L, L)]
        take = jnp.clip(nvec - jnp.int32(j), 0, 1)
        sumvec = sumvec + take * nonzero_mask(v)
    prefix = splat_last(scan16(sumvec))

    carry = prefix
    for j in range(VPT):
        v = chunk_v[pl.ds(j * L, L)]
        mi = nonzero_mask(v)
        s = scan16(mi)
        pos = (carry + s) * mi
        idx_v[j // (R // L), pl.ds((j % (R // L)) * L, L)] = pos
        carry = carry + splat_last(s)

    base = wid * PER
    gsems = (gsem0, gsem1)
    osems = (osem0, osem1)

    def gather(k):
        return pltpu.async_copy(
            tab_hbm.at[idx_v.at[k]], buf_v.at[k % NB], gsems[k % NB])

    def outcopy(k):
        return pltpu.async_copy(
            buf_v.at[k % NB], out_hbm.at[pl.ds(base + k * R, R)],
            osems[k % NB])

    gs = [gather(0)]
    os_ = []
    for k in range(NCH):
        gs[k].wait()
        os_.append(outcopy(k))
        if k + 1 < NCH:
            if k >= 1:
                os_[k - 1].wait()
            gs.append(gather(k + 1))
    os_[NCH - 2].wait()
    os_[NCH - 1].wait()


@jax.jit
def _sc_embed(flat_inp, table):
    mesh = plsc.VectorSubcoreMesh(
        core_axis_name="c", subcore_axis_name="s", num_cores=NC, num_subcores=NS
    )
    return pl.kernel(
        _body,
        out_type=jax.ShapeDtypeStruct((B * T, D), jnp.float32),
        mesh=mesh,
        scratch_types=[
            pltpu.VMEM((T,), jnp.int32),
            pltpu.VMEM((PER,), jnp.int32),
            pltpu.VMEM((NCH, R), jnp.int32),
            pltpu.VMEM((NB, R, D), jnp.float32),
            pltpu.SemaphoreType.DMA,
            pltpu.SemaphoreType.DMA,
            pltpu.SemaphoreType.DMA,
            pltpu.SemaphoreType.DMA,
        ],
    )(flat_inp, table)


def kernel(input, embeddings):
    flat = input.reshape(-1).astype(jnp.int32)
    out = _sc_embed(flat, embeddings.astype(jnp.float32))
    return out.reshape(B, T, D)


# speedup vs baseline: 1.5307x; 1.0418x over previous
import functools

import jax
import jax.numpy as jnp
from jax import lax
from jax.experimental import pallas as pl
from jax.experimental.pallas import tpu as pltpu
from jax.experimental.pallas import tpu_sc as plsc

B = 4
T = 2048
D = 1024
L = 16
NC = 2
NS = 16
NW = NC * NS
PER = (B * T) // NW
VPT = PER // L
VPR = T // L
R = 32
NCH = PER // R
NB = 3


def _body(inp_hbm, tab_hbm, out_hbm, row_v, chunk_v, idx_v, buf_v,
          gsem0, gsem1, gsem2, osem0, osem1, osem2):
    wid = lax.axis_index("s") * NC + lax.axis_index("c")
    row = wid // (T // PER)
    ch = wid % (T // PER)

    pltpu.sync_copy(inp_hbm.at[pl.ds(row * T, T)], row_v)
    pltpu.sync_copy(inp_hbm.at[pl.ds(wid * PER, PER)], chunk_v)

    zeros = jnp.zeros((L,), jnp.int32)
    lane = lax.iota(jnp.int32, L)
    last = jnp.broadcast_to(jnp.int32(L - 1), (L,))

    def nonzero_mask(v):
        return lax.shift_right_logical(v | (zeros - v), 31)

    def scan16(x):
        s = x
        for d in (1, 2, 4, 8):
            shifted = s.at[jnp.maximum(lane - d, 0)].get(mode="promise_in_bounds")
            s = s + jnp.clip(lane - (d - 1), 0, 1) * shifted
        return s

    def splat_last(s):
        return s.at[last].get(mode="promise_in_bounds")

    nvec = jnp.broadcast_to(ch * (PER // L), (L,))
    sumvec = zeros
    for j in range(VPR):
        v = row_v[pl.ds(j * L, L)]
        take = jnp.clip(nvec - jnp.int32(j), 0, 1)
        sumvec = sumvec + take * nonzero_mask(v)
    prefix = splat_last(scan16(sumvec))

    carry = prefix
    for j in range(VPT):
        v = chunk_v[pl.ds(j * L, L)]
        mi = nonzero_mask(v)
        s = scan16(mi)
        pos = (carry + s) * mi
        idx_v[j // (R // L), pl.ds((j % (R // L)) * L, L)] = pos
        carry = carry + splat_last(s)

    base = wid * PER
    gsems = (gsem0, gsem1, gsem2)
    osems = (osem0, osem1, osem2)

    def gather(k):
        return pltpu.async_copy(
            tab_hbm.at[idx_v.at[k]], buf_v.at[k % NB], gsems[k % NB])

    def outcopy(k):
        return pltpu.async_copy(
            buf_v.at[k % NB], out_hbm.at[pl.ds(base + k * R, R)],
            osems[k % NB])

    gs = [gather(0), gather(1)]
    os_ = []
    for k in range(NCH):
        gs[k].wait()
        os_.append(outcopy(k))
        if k + 2 < NCH:
            if k >= 1:
                os_[k - 1].wait()
            gs.append(gather(k + 2))
    os_[NCH - 3].wait()
    os_[NCH - 2].wait()
    os_[NCH - 1].wait()


@jax.jit
def _sc_embed(flat_inp, table):
    mesh = plsc.VectorSubcoreMesh(
        core_axis_name="c", subcore_axis_name="s", num_cores=NC, num_subcores=NS
    )
    return pl.kernel(
        _body,
        out_type=jax.ShapeDtypeStruct((B * T, D), jnp.float32),
        mesh=mesh,
        scratch_types=[
            pltpu.VMEM((T,), jnp.int32),
            pltpu.VMEM((PER,), jnp.int32),
            pltpu.VMEM((NCH, R), jnp.int32),
            pltpu.VMEM((NB, R, D), jnp.float32),
            pltpu.SemaphoreType.DMA,
            pltpu.SemaphoreType.DMA,
            pltpu.SemaphoreType.DMA,
            pltpu.SemaphoreType.DMA,
            pltpu.SemaphoreType.DMA,
            pltpu.SemaphoreType.DMA,
        ],
    )(flat_inp, table)


def kernel(input, embeddings):
    flat = input.reshape(-1).astype(jnp.int32)
    out = _sc_embed(flat, embeddings.astype(jnp.float32))
    return out.reshape(B, T, D)
